# Initial kernel scaffold; baseline (speedup 1.0000x reference)
import functools

import jax
import jax.numpy as jnp
from jax import lax
from jax.experimental import pallas as pl
from jax.experimental.pallas import tpu as pltpu
from jax.experimental.pallas import tpu_sc as plsc

N_POINTS = 4096
M1 = 2048
M2 = 512
K_NBR = 64
R1SQ = 0.2 * 0.2
R2SQ = 0.4 * 0.4

_F32 = jnp.float32
_INF = jnp.float32(jnp.inf)
_IMAX = jnp.int32(2147483647)
_PREC = jax.lax.Precision.HIGHEST


def _dot(a, b):
    return jnp.dot(a, b, preferred_element_type=_F32, precision=_PREC)


def _fps_body(pos_ref, idx_ref, sel_ref):
    x = pos_ref[0]
    y = pos_ref[1]
    z = pos_ref[2]
    nr = x.shape[0]
    mr = idx_ref.shape[0]
    iota_n = (lax.broadcasted_iota(jnp.int32, (nr, 128), 0) * 128
              + lax.broadcasted_iota(jnp.int32, (nr, 128), 1))
    iota_m = (lax.broadcasted_iota(jnp.int32, (mr, 128), 0) * 128
              + lax.broadcasted_iota(jnp.int32, (mr, 128), 1))
    m = mr * 128

    cx0 = x[0, 0]
    cy0 = y[0, 0]
    cz0 = z[0, 0]
    dists0 = jnp.full((nr, 128), _INF, _F32)
    idxs0 = jnp.zeros((mr, 128), jnp.int32)
    sx0 = jnp.where(iota_m == 0, cx0, 0.0).astype(_F32)
    sy0 = jnp.where(iota_m == 0, cy0, 0.0).astype(_F32)
    sz0 = jnp.where(iota_m == 0, cz0, 0.0).astype(_F32)

    def body(i, st):
        dists, idxs, sx, sy, sz, cx, cy, cz = st
        dd = (x - cx) ** 2 + (y - cy) ** 2 + (z - cz) ** 2
        dists = jnp.minimum(dists, dd)
        mx = jnp.max(dists)
        cand = jnp.where(dists == mx, iota_n, _IMAX)
        idx = jnp.min(cand)
        selmask = iota_n == idx
        cx = jnp.sum(jnp.where(selmask, x, 0.0))
        cy = jnp.sum(jnp.where(selmask, y, 0.0))
        cz = jnp.sum(jnp.where(selmask, z, 0.0))
        here = iota_m == i
        idxs = jnp.where(here, idx, idxs)
        sx = jnp.where(here, cx, sx)
        sy = jnp.where(here, cy, sy)
        sz = jnp.where(here, cz, sz)
        return (dists, idxs, sx, sy, sz, cx, cy, cz)

    st = lax.fori_loop(1, m, body,
                       (dists0, idxs0, sx0, sy0, sz0, cx0, cy0, cz0))
    _, idxs, sx, sy, sz, _, _, _ = st
    idx_ref[...] = idxs
    sel_ref[0] = sx
    sel_ref[1] = sy
    sel_ref[2] = sz


def _fps(pos, m):
    n = pos.shape[0]
    pos_r = pos.T.reshape(3, n // 128, 128)
    idx_r, sel_r = pl.pallas_call(
        _fps_body,
        out_shape=[
            jax.ShapeDtypeStruct((m // 128, 128), jnp.int32),
            jax.ShapeDtypeStruct((3, m // 128, 128), _F32),
        ],
    )(pos_r)
    return idx_r.reshape(m), sel_r.reshape(3, m).T


def _knn_body(a_ref, bt_ref, nbr_ref, val_ref, *, k):
    a = a_ref[...]
    bt = bt_ref[...]
    n = bt.shape[1]
    rb = a.shape[0]
    aa = jnp.sum(a * a, axis=1, keepdims=True)
    bb = jnp.sum(bt * bt, axis=0, keepdims=True)
    d2 = jnp.maximum(aa + bb - 2.0 * _dot(a, bt), 0.0)
    iota_c = lax.broadcasted_iota(jnp.int32, (rb, n), 1)
    iota_k = lax.broadcasted_iota(jnp.int32, (rb, k), 1)

    def body(j, st):
        d2, nbrs, vals = st
        rowmin = jnp.min(d2, axis=1, keepdims=True)
        cand = jnp.where(d2 == rowmin, iota_c, _IMAX)
        idx = jnp.min(cand, axis=1, keepdims=True)
        chosen = cand == idx
        d2 = jnp.where(chosen, _INF, d2)
        here = iota_k == j
        nbrs = jnp.where(here, idx, nbrs)
        vals = jnp.where(here, rowmin, vals)
        return (d2, nbrs, vals)

    nbrs0 = jnp.zeros((rb, k), jnp.int32)
    vals0 = jnp.zeros((rb, k), _F32)
    _, nbrs, vals = lax.fori_loop(0, k, body, (d2, nbrs0, vals0))
    nbr_ref[...] = nbrs
    val_ref[...] = vals


def _knn(a, b, k, rb):
    m = a.shape[0]
    n = b.shape[0]
    bt = b.T
    grid = m // rb
    return pl.pallas_call(
        functools.partial(_knn_body, k=k),
        grid=(grid,),
        in_specs=[
            pl.BlockSpec((rb, 3), lambda i: (i, 0)),
            pl.BlockSpec((3, n), lambda i: (0, 0)),
        ],
        out_specs=[
            pl.BlockSpec((rb, k), lambda i: (i, 0)),
            pl.BlockSpec((rb, k), lambda i: (i, 0)),
        ],
        out_shape=[
            jax.ShapeDtypeStruct((m, k), jnp.int32),
            jax.ShapeDtypeStruct((m, k), _F32),
        ],
    )(a, bt)


_SC_CHUNK = 128


def _sc_gather(table, idx):
    v, d = table.shape
    b = idx.shape[0]
    nw = 32
    b_per_w = b // nw
    nch = b_per_w // _SC_CHUNK
    mesh = plsc.VectorSubcoreMesh(core_axis_name="c", subcore_axis_name="s")

    @functools.partial(
        pl.kernel,
        out_type=jax.ShapeDtypeStruct((b, d), _F32),
        mesh=mesh,
        scratch_types=[
            pltpu.VMEM((_SC_CHUNK,), jnp.int32),
            pltpu.VMEM((_SC_CHUNK, d), _F32),
            pltpu.SemaphoreType.DMA,
        ],
    )
    def gk(table_hbm, idx_hbm, out_hbm, idx_v, rows_v, sem):
        wid = lax.axis_index("s") * 2 + lax.axis_index("c")
        base = wid * b_per_w

        def chunk(i, carry):
            off = base + i * _SC_CHUNK
            pltpu.sync_copy(idx_hbm.at[pl.ds(off, _SC_CHUNK)], idx_v)
            pltpu.async_copy(table_hbm.at[idx_v], rows_v, sem).wait()
            pltpu.sync_copy(rows_v, out_hbm.at[pl.ds(off, _SC_CHUNK)])
            return carry

        lax.fori_loop(0, nch, chunk, 0)

    return gk(table, idx)


def _sa_body(g_ref, s_ref, v_ref, w1_ref, b1_ref, w2_ref, b2_ref,
             w3_ref, b3_ref, o_ref, *, k, rsq):
    g = g_ref[...]
    s = s_ref[...]
    rb = s.shape[0]
    dp = s.shape[1]
    feats = (g.reshape(rb, k, dp) - s[:, None, :]).reshape(rb * k, dp)
    h = jnp.maximum(_dot(feats, w1_ref[...]) + b1_ref[...], 0.0)
    h = jnp.maximum(_dot(h, w2_ref[...]) + b2_ref[...], 0.0)
    h = _dot(h, w3_ref[...]) + b3_ref[...]
    d3 = h.shape[1]
    valid = v_ref[...] <= rsq
    h3 = jnp.where(valid[:, :, None], h.reshape(rb, k, d3), -_INF)
    o_ref[...] = jnp.max(h3, axis=1)


def _sa_mlp(gath, sub, vals, w1, b1, w2, b2, w3, b3, rsq, rb):
    m, dp = sub.shape
    k = vals.shape[1]
    d1 = w1.shape[1]
    d2d = w2.shape[1]
    d3 = w3.shape[1]
    grid = m // rb
    full = lambda shape: pl.BlockSpec(shape, lambda i: tuple(0 for _ in shape))
    return pl.pallas_call(
        functools.partial(_sa_body, k=k, rsq=rsq),
        grid=(grid,),
        in_specs=[
            pl.BlockSpec((rb * k, dp), lambda i: (i, 0)),
            pl.BlockSpec((rb, dp), lambda i: (i, 0)),
            pl.BlockSpec((rb, k), lambda i: (i, 0)),
            full((dp, d1)), full((1, d1)),
            full((d1, d2d)), full((1, d2d)),
            full((d2d, d3)), full((1, d3)),
        ],
        out_specs=pl.BlockSpec((rb, d3), lambda i: (i, 0)),
        out_shape=jax.ShapeDtypeStruct((m, d3), _F32),
    )(gath, sub, vals, w1, b1, w2, b2, w3, b3)


def _global_body(x2_ref, p2_ref, wa_ref, wb_ref, b1_ref, w2_ref, b2_ref,
                 w3_ref, b3_ref, ua_ref, ub_ref, ub_b_ref, o_ref):
    x2 = x2_ref[...]
    p2 = p2_ref[...]
    h = jnp.maximum(_dot(x2, wa_ref[...]) + _dot(p2, wb_ref[...])
                    + b1_ref[...], 0.0)
    h = jnp.maximum(_dot(h, w2_ref[...]) + b2_ref[...], 0.0)
    h = _dot(h, w3_ref[...]) + b3_ref[...]
    x3 = jnp.max(h, axis=0, keepdims=True)
    up2 = (_dot(x2, ua_ref[...]) + _dot(x3, ub_ref[...])
           + ub_b_ref[...])
    o_ref[...] = up2


def _interp_weights(pa, pbt):
    n = pbt.shape[1]
    rb = pa.shape[0]
    aa = jnp.sum(pa * pa, axis=1, keepdims=True)
    bb = jnp.sum(pbt * pbt, axis=0, keepdims=True)
    d2 = jnp.maximum(aa + bb - 2.0 * _dot(pa, pbt), 0.0)
    iota_c = lax.broadcasted_iota(jnp.int32, (rb, n), 1)
    wmat = jnp.zeros((rb, n), _F32)
    wsum = jnp.zeros((rb, 1), _F32)
    for _ in range(3):
        rowmin = jnp.min(d2, axis=1, keepdims=True)
        cand = jnp.where(d2 == rowmin, iota_c, _IMAX)
        idx = jnp.min(cand, axis=1, keepdims=True)
        chosen = cand == idx
        d2 = jnp.where(chosen, _INF, d2)
        wj = 1.0 / (rowmin + 1e-8)
        wmat = wmat + jnp.where(chosen, wj, 0.0)
        wsum = wsum + wj
    return wmat / wsum


def _up1_body(p1_ref, p2t_ref, up2_ref, x1_ref, ua_ref, ub_ref, b_ref, o_ref):
    wmat = _interp_weights(p1_ref[...], p2t_ref[...])
    feat = _dot(wmat, up2_ref[...])
    o_ref[...] = (_dot(x1_ref[...], ua_ref[...]) + _dot(feat, ub_ref[...])
                  + b_ref[...])


def _seg1_body(p_ref, p1t_ref, up1_ref, w_ref, b_ref, o_ref):
    wmat = _interp_weights(p_ref[...], p1t_ref[...])
    feat = _dot(wmat, up1_ref[...])
    o_ref[...] = _dot(feat, w_ref[...]) + b_ref[...]


def _head_body(h_ref, g_ref, be_ref, w_ref, b_ref, o_ref):
    h = h_ref[...]
    n = h.shape[0]
    mean = jnp.sum(h, axis=0, keepdims=True) / n
    cen = h - mean
    var = jnp.sum(cen * cen, axis=0, keepdims=True) / n
    hb = g_ref[...] * cen / jnp.sqrt(var + 1e-5) + be_ref[...]
    hb = jnp.maximum(hb, 0.0)
    o_ref[...] = _dot(hb, w_ref[...]) + b_ref[...]


def kernel(pos, batch, params):
    del batch
    pos = pos.astype(_F32)

    idx1, pos1 = _fps(pos, M1)
    idx2, pos2 = _fps(pos1, M2)
    del idx1, idx2

    nbr1, d2n1 = _knn(pos1, pos, K_NBR, 128)
    nbr2, d2n2 = _knn(pos2, pos1, K_NBR, 128)

    pos_pad = jnp.pad(pos, ((0, 0), (0, 13)))
    g1 = _sc_gather(pos_pad, nbr1.reshape(M1 * K_NBR))

    p = params
    w1a = jnp.zeros((16, 64), _F32).at[:3].set(p["mlp1"][0][0])
    sa1_sub = jnp.pad(pos1, ((0, 0), (0, 13)))
    x1 = _sa_mlp(g1, sa1_sub, d2n1,
                 w1a, p["mlp1"][0][1][None, :],
                 p["mlp1"][1][0], p["mlp1"][1][1][None, :],
                 p["mlp1"][2][0], p["mlp1"][2][1][None, :],
                 R1SQ, 128)

    feat1 = jnp.concatenate(
        [x1, pos1, jnp.zeros((M1, 13), _F32)], axis=1)
    g2 = _sc_gather(feat1, nbr2.reshape(M2 * K_NBR))
    w2a = jnp.zeros((144, 128), _F32).at[:131].set(p["mlp2"][0][0])
    sa2_sub = jnp.concatenate(
        [jnp.zeros((M2, 128), _F32), pos2, jnp.zeros((M2, 13), _F32)], axis=1)
    x2 = _sa_mlp(g2, sa2_sub, d2n2,
                 w2a, p["mlp2"][0][1][None, :],
                 p["mlp2"][1][0], p["mlp2"][1][1][None, :],
                 p["mlp2"][2][0], p["mlp2"][2][1][None, :],
                 R2SQ, 64)

    m3 = p["mlp3"]
    up2 = pl.pallas_call(_global_body, out_shape=jax.ShapeDtypeStruct(
        (M2, 512), _F32))(
        x2, pos2,
        m3[0][0][:256], m3[0][0][256:259], m3[0][1][None, :],
        m3[1][0], m3[1][1][None, :],
        m3[2][0], m3[2][1][None, :],
        p["upconv1"][0][:256], p["upconv1"][0][256:1280],
        p["upconv1"][1][None, :])

    up1 = pl.pallas_call(
        _up1_body,
        grid=(M1 // 512,),
        in_specs=[
            pl.BlockSpec((512, 3), lambda i: (i, 0)),
            pl.BlockSpec((3, M2), lambda i: (0, 0)),
            pl.BlockSpec((M2, 512), lambda i: (0, 0)),
            pl.BlockSpec((512, 128), lambda i: (i, 0)),
            pl.BlockSpec((128, 256), lambda i: (0, 0)),
            pl.BlockSpec((512, 256), lambda i: (0, 0)),
            pl.BlockSpec((1, 256), lambda i: (0, 0)),
        ],
        out_specs=pl.BlockSpec((512, 256), lambda i: (i, 0)),
        out_shape=jax.ShapeDtypeStruct((M1, 256), _F32),
    )(pos1, pos2.T, up2, x1,
      p["upconv2"][0][:128], p["upconv2"][0][128:640],
      p["upconv2"][1][None, :])

    h = pl.pallas_call(
        _seg1_body,
        grid=(N_POINTS // 512,),
        in_specs=[
            pl.BlockSpec((512, 3), lambda i: (i, 0)),
            pl.BlockSpec((3, M1), lambda i: (0, 0)),
            pl.BlockSpec((M1, 256), lambda i: (0, 0)),
            pl.BlockSpec((256, 512), lambda i: (0, 0)),
            pl.BlockSpec((1, 512), lambda i: (0, 0)),
        ],
        out_specs=pl.BlockSpec((512, 512), lambda i: (i, 0)),
        out_shape=jax.ShapeDtypeStruct((N_POINTS, 512), _F32),
    )(pos, pos1.T, up1, p["seg1"][0], p["seg1"][1][None, :])

    w2p = jnp.zeros((512, 128), _F32).at[:, :13].set(p["seg2"][0])
    b2p = jnp.zeros((1, 128), _F32).at[:, :13].set(p["seg2"][1])
    logits_pad = pl.pallas_call(_head_body, out_shape=jax.ShapeDtypeStruct(
        (N_POINTS, 128), _F32))(
        h, p["bn_gamma"][None, :], p["bn_beta"][None, :], w2p, b2p)

    logits = logits_pad[:, :13]
    return jnp.transpose(logits)[None, :, :]


# speedup vs baseline: 6.0507x; 6.0507x over previous
import functools

import jax
import jax.numpy as jnp
from jax import lax
from jax.experimental import pallas as pl
from jax.experimental.pallas import tpu as pltpu
from jax.experimental.pallas import tpu_sc as plsc

N_POINTS = 4096
M1 = 2048
M2 = 512
K_NBR = 64
R1SQ = 0.2 * 0.2
R2SQ = 0.4 * 0.4

_F32 = jnp.float32
_INF = float("inf")
_IMAX = 2147483647
_PREC = jax.lax.Precision.HIGHEST


def _dot(a, b):
    return jnp.dot(a.astype(jnp.bfloat16), b.astype(jnp.bfloat16),
                   preferred_element_type=_F32)


def _dotf(a, b):
    return jnp.dot(a, b, preferred_element_type=_F32, precision=_PREC)


def _fps_body(pos_ref, idx_ref, sel_ref):
    x = pos_ref[0]
    y = pos_ref[1]
    z = pos_ref[2]
    nr = x.shape[0]
    mr = idx_ref.shape[0]
    iota_n = (lax.broadcasted_iota(jnp.int32, (nr, 128), 0) * 128
              + lax.broadcasted_iota(jnp.int32, (nr, 128), 1))
    iota_m = (lax.broadcasted_iota(jnp.int32, (mr, 128), 0) * 128
              + lax.broadcasted_iota(jnp.int32, (mr, 128), 1))
    m = mr * 128

    cx0 = x[0, 0]
    cy0 = y[0, 0]
    cz0 = z[0, 0]
    dists0 = jnp.full((nr, 128), _INF, _F32)
    idxs0 = jnp.zeros((mr, 128), jnp.int32)
    sx0 = jnp.where(iota_m == 0, cx0, 0.0).astype(_F32)
    sy0 = jnp.where(iota_m == 0, cy0, 0.0).astype(_F32)
    sz0 = jnp.where(iota_m == 0, cz0, 0.0).astype(_F32)

    def body(i, st):
        dists, idxs, sx, sy, sz, cx, cy, cz = st
        dd = (x - cx) ** 2 + (y - cy) ** 2 + (z - cz) ** 2
        dists = jnp.minimum(dists, dd)
        mx = jnp.max(dists)
        cand = jnp.where(dists == mx, iota_n, _IMAX)
        idx = jnp.min(cand)
        selmask = iota_n == idx
        cx = jnp.sum(jnp.where(selmask, x, 0.0))
        cy = jnp.sum(jnp.where(selmask, y, 0.0))
        cz = jnp.sum(jnp.where(selmask, z, 0.0))
        here = iota_m == i
        idxs = jnp.where(here, idx, idxs)
        sx = jnp.where(here, cx, sx)
        sy = jnp.where(here, cy, sy)
        sz = jnp.where(here, cz, sz)
        return (dists, idxs, sx, sy, sz, cx, cy, cz)

    st = lax.fori_loop(1, m, body,
                       (dists0, idxs0, sx0, sy0, sz0, cx0, cy0, cz0))
    _, idxs, sx, sy, sz, _, _, _ = st
    idx_ref[...] = idxs
    sel_ref[0] = sx
    sel_ref[1] = sy
    sel_ref[2] = sz


def _fps(pos, m):
    n = pos.shape[0]
    pos_r = pos.T.reshape(3, n // 128, 128)
    idx_r, sel_r = pl.pallas_call(
        _fps_body,
        out_shape=[
            jax.ShapeDtypeStruct((m // 128, 128), jnp.int32),
            jax.ShapeDtypeStruct((3, m // 128, 128), _F32),
        ],
    )(pos_r)
    return idx_r.reshape(m), sel_r.reshape(3, m).T


def _sqdist_block(a, bt, aa, bb):
    return jnp.maximum(aa + bb - 2.0 * _dot(a, bt), 0.0)


def _knn_body(a_ref, bt_ref, an_ref, bn_ref, nbr_ref, val_ref, *, k):
    a = a_ref[...]
    bt = bt_ref[...]
    n = bt.shape[1]
    rb = a.shape[0]
    d2 = _sqdist_block(a, bt, an_ref[...], bn_ref[...])
    iota_c = lax.broadcasted_iota(jnp.int32, (rb, n), 1)
    iota_k = lax.broadcasted_iota(jnp.int32, (rb, k), 1)

    def body(j, st):
        d2, nbrs, vals = st
        rowmin = jnp.min(d2, axis=1, keepdims=True)
        cand = jnp.where(d2 == rowmin, iota_c, _IMAX)
        idx = jnp.min(cand, axis=1, keepdims=True)
        chosen = cand == idx
        d2 = jnp.where(chosen, _INF, d2)
        here = iota_k == j
        nbrs = jnp.where(here, idx, nbrs)
        vals = jnp.where(here, rowmin, vals)
        return (d2, nbrs, vals)

    nbrs0 = jnp.zeros((rb, k), jnp.int32)
    vals0 = jnp.zeros((rb, k), _F32)
    _, nbrs, vals = lax.fori_loop(0, k, body, (d2, nbrs0, vals0))
    nbr_ref[...] = nbrs
    val_ref[...] = vals


def _norms(x):
    return jnp.sum(x * x, axis=-1)


def _knn(a, b, k, rb):
    m = a.shape[0]
    n = b.shape[0]
    bt = b.T
    an = _norms(a)[:, None]
    bn = _norms(b)[None, :]
    grid = m // rb
    return pl.pallas_call(
        functools.partial(_knn_body, k=k),
        grid=(grid,),
        in_specs=[
            pl.BlockSpec((rb, 3), lambda i: (i, 0)),
            pl.BlockSpec((3, n), lambda i: (0, 0)),
            pl.BlockSpec((rb, 1), lambda i: (i, 0)),
            pl.BlockSpec((1, n), lambda i: (0, 0)),
        ],
        out_specs=[
            pl.BlockSpec((rb, k), lambda i: (i, 0)),
            pl.BlockSpec((rb, k), lambda i: (i, 0)),
        ],
        out_shape=[
            jax.ShapeDtypeStruct((m, k), jnp.int32),
            jax.ShapeDtypeStruct((m, k), _F32),
        ],
    )(a, bt, an, bn)


_SC_CHUNK = 128


def _sc_gather(table, idx):
    v, d = table.shape
    b = idx.shape[0]
    nw = 32
    b_per_w = b // nw
    nch = b_per_w // _SC_CHUNK
    mesh = plsc.VectorSubcoreMesh(core_axis_name="c", subcore_axis_name="s")

    @functools.partial(
        pl.kernel,
        out_type=jax.ShapeDtypeStruct((b, d), _F32),
        mesh=mesh,
        scratch_types=[
            pltpu.VMEM((_SC_CHUNK,), jnp.int32),
            pltpu.VMEM((_SC_CHUNK, d), _F32),
            pltpu.SemaphoreType.DMA,
        ],
        compiler_params=pltpu.CompilerParams(use_tc_tiling_on_sc=False),
    )
    def gk(table_hbm, idx_hbm, out_hbm, idx_v, rows_v, sem):
        wid = lax.axis_index("s") * 2 + lax.axis_index("c")
        base = wid * b_per_w

        def chunk(i, carry):
            off = base + i * _SC_CHUNK
            pltpu.sync_copy(idx_hbm.at[pl.ds(off, _SC_CHUNK)], idx_v)
            pltpu.async_copy(table_hbm.at[idx_v], rows_v, sem).wait()
            pltpu.sync_copy(rows_v, out_hbm.at[pl.ds(off, _SC_CHUNK)])
            return carry

        lax.fori_loop(0, nch, chunk, 0)

    return gk(table, idx)


def _sa_body(g_ref, s_ref, v_ref, w1_ref, b1_ref, w2_ref, b2_ref,
             w3_ref, b3_ref, o_ref, *, k, rsq):
    g = g_ref[...]
    s = s_ref[...]
    rb = s.shape[0]
    dp = s.shape[1]
    feats = (g.reshape(rb, k, dp) - s[:, None, :]).reshape(rb * k, dp)
    h = jnp.maximum(_dot(feats, w1_ref[...]) + b1_ref[...], 0.0)
    h = jnp.maximum(_dot(h, w2_ref[...]) + b2_ref[...], 0.0)
    h = _dot(h, w3_ref[...]) + b3_ref[...]
    d3 = h.shape[1]
    penalty = jnp.where(v_ref[...] <= rsq, 0.0, -_INF)
    h = h + penalty
    o_ref[...] = jnp.max(h.reshape(rb, k, d3), axis=1)


def _sa_mlp(gath, sub, vals, w1, b1, w2, b2, w3, b3, rsq, rb):
    m, dp = sub.shape
    k = vals.shape[1]
    d1 = w1.shape[1]
    d2d = w2.shape[1]
    d3 = w3.shape[1]
    grid = m // rb
    vals_col = vals.reshape(m * k, 1)
    full = lambda shape: pl.BlockSpec(shape, lambda i: tuple(0 for _ in shape))
    return pl.pallas_call(
        functools.partial(_sa_body, k=k, rsq=rsq),
        grid=(grid,),
        in_specs=[
            pl.BlockSpec((rb * k, dp), lambda i: (i, 0)),
            pl.BlockSpec((rb, dp), lambda i: (i, 0)),
            pl.BlockSpec((rb * k, 1), lambda i: (i, 0)),
            full((dp, d1)), full((1, d1)),
            full((d1, d2d)), full((1, d2d)),
            full((d2d, d3)), full((1, d3)),
        ],
        out_specs=pl.BlockSpec((rb, d3), lambda i: (i, 0)),
        out_shape=jax.ShapeDtypeStruct((m, d3), _F32),
    )(gath, sub, vals_col, w1, b1, w2, b2, w3, b3)


def _global_body(x2_ref, p2_ref, w1_ref, b1_ref, w2_ref, b2_ref,
                 w3_ref, b3_ref, u_ref, ub_ref, o_ref):
    x2 = x2_ref[...]
    p2 = p2_ref[...]
    cat = jnp.concatenate([x2, p2], axis=1)
    h = jnp.maximum(_dot(cat, w1_ref[...]) + b1_ref[...], 0.0)
    h = jnp.maximum(_dot(h, w2_ref[...]) + b2_ref[...], 0.0)
    h = _dot(h, w3_ref[...]) + b3_ref[...]
    x3 = jnp.max(h, axis=0, keepdims=True)
    m2 = x2.shape[0]
    cat2 = jnp.concatenate(
        [x2, jnp.broadcast_to(x3, (m2, x3.shape[1]))], axis=1)
    o_ref[...] = _dot(cat2, u_ref[...]) + ub_ref[...]


def _interp_feat(pa, pbt, an, bn, up):
    n = pbt.shape[1]
    rb = pa.shape[0]
    d2 = _sqdist_block(pa, pbt, an, bn)
    iota_c = lax.broadcasted_iota(jnp.int32, (rb, n), 1)
    sel = []
    wsum = jnp.zeros((rb, 1), _F32)
    for _ in range(3):
        rowmin = jnp.min(d2, axis=1, keepdims=True)
        cand = jnp.where(d2 == rowmin, iota_c, _IMAX)
        idx = jnp.min(cand, axis=1, keepdims=True)
        chosen = cand == idx
        d2 = jnp.where(chosen, _INF, d2)
        wj = 1.0 / (rowmin + 1e-8)
        sel.append((chosen, wj))
        wsum = wsum + wj
    feat = None
    for chosen, wj in sel:
        fj = _dotf(chosen.astype(_F32), up)
        term = (wj / wsum) * fj
        feat = term if feat is None else feat + term
    return feat


def _up1_body(p1_ref, p2t_ref, an_ref, bn_ref, up2_ref, x1_ref,
              u_ref, b_ref, o_ref):
    feat = _interp_feat(p1_ref[...], p2t_ref[...],
                        an_ref[...], bn_ref[...], up2_ref[...])
    cat = jnp.concatenate([x1_ref[...], feat], axis=1)
    o_ref[...] = _dot(cat, u_ref[...]) + b_ref[...]


def _seg1_body(p_ref, p1t_ref, an_ref, bn_ref, up1_ref, w_ref, b_ref, o_ref):
    feat = _interp_feat(p_ref[...], p1t_ref[...],
                        an_ref[...], bn_ref[...], up1_ref[...])
    o_ref[...] = _dot(feat, w_ref[...]) + b_ref[...]


def _head_body(h_ref, g_ref, be_ref, w_ref, b_ref, o_ref):
    h = h_ref[...]
    n = h.shape[0]
    mean = jnp.sum(h, axis=0, keepdims=True) / n
    cen = h - mean
    var = jnp.sum(cen * cen, axis=0, keepdims=True) / n
    hb = g_ref[...] * cen / jnp.sqrt(var + 1e-5) + be_ref[...]
    hb = jnp.maximum(hb, 0.0)
    o_ref[...] = _dot(hb, w_ref[...]) + b_ref[...]


def kernel(pos, batch, params):
    del batch
    pos = pos.astype(_F32)

    idx1, pos1 = _fps(pos, M1)
    idx2, pos2 = _fps(pos1, M2)
    del idx1, idx2

    nbr1, d2n1 = _knn(pos1, pos, K_NBR, 128)
    nbr2, d2n2 = _knn(pos2, pos1, K_NBR, 128)

    pos_pad = jnp.pad(pos, ((0, 0), (0, 13)))
    g1 = _sc_gather(pos_pad, nbr1.reshape(M1 * K_NBR))

    p = params
    w1a = jnp.zeros((16, 64), _F32).at[:3].set(p["mlp1"][0][0])
    sa1_sub = jnp.pad(pos1, ((0, 0), (0, 13)))
    x1 = _sa_mlp(g1, sa1_sub, d2n1,
                 w1a, p["mlp1"][0][1][None, :],
                 p["mlp1"][1][0], p["mlp1"][1][1][None, :],
                 p["mlp1"][2][0], p["mlp1"][2][1][None, :],
                 R1SQ, 128)

    feat1 = jnp.concatenate(
        [x1, pos1, jnp.zeros((M1, 13), _F32)], axis=1)
    g2 = _sc_gather(feat1, nbr2.reshape(M2 * K_NBR))
    w2a = jnp.zeros((144, 128), _F32).at[:131].set(p["mlp2"][0][0])
    sa2_sub = jnp.concatenate(
        [jnp.zeros((M2, 128), _F32), pos2, jnp.zeros((M2, 13), _F32)], axis=1)
    x2 = _sa_mlp(g2, sa2_sub, d2n2,
                 w2a, p["mlp2"][0][1][None, :],
                 p["mlp2"][1][0], p["mlp2"][1][1][None, :],
                 p["mlp2"][2][0], p["mlp2"][2][1][None, :],
                 R2SQ, 64)

    m3 = p["mlp3"]
    up2 = pl.pallas_call(_global_body, out_shape=jax.ShapeDtypeStruct(
        (M2, 512), _F32))(
        x2, pos2,
        m3[0][0], m3[0][1][None, :],
        m3[1][0], m3[1][1][None, :],
        m3[2][0], m3[2][1][None, :],
        p["upconv1"][0], p["upconv1"][1][None, :])

    up1 = pl.pallas_call(
        _up1_body,
        grid=(M1 // 512,),
        in_specs=[
            pl.BlockSpec((512, 3), lambda i: (i, 0)),
            pl.BlockSpec((3, M2), lambda i: (0, 0)),
            pl.BlockSpec((512, 1), lambda i: (i, 0)),
            pl.BlockSpec((1, M2), lambda i: (0, 0)),
            pl.BlockSpec((M2, 512), lambda i: (0, 0)),
            pl.BlockSpec((512, 128), lambda i: (i, 0)),
            pl.BlockSpec((640, 256), lambda i: (0, 0)),
            pl.BlockSpec((1, 256), lambda i: (0, 0)),
        ],
        out_specs=pl.BlockSpec((512, 256), lambda i: (i, 0)),
        out_shape=jax.ShapeDtypeStruct((M1, 256), _F32),
    )(pos1, pos2.T, _norms(pos1)[:, None], _norms(pos2)[None, :], up2, x1,
      p["upconv2"][0], p["upconv2"][1][None, :])

    h = pl.pallas_call(
        _seg1_body,
        grid=(N_POINTS // 512,),
        in_specs=[
            pl.BlockSpec((512, 3), lambda i: (i, 0)),
            pl.BlockSpec((3, M1), lambda i: (0, 0)),
            pl.BlockSpec((512, 1), lambda i: (i, 0)),
            pl.BlockSpec((1, M1), lambda i: (0, 0)),
            pl.BlockSpec((M1, 256), lambda i: (0, 0)),
            pl.BlockSpec((256, 512), lambda i: (0, 0)),
            pl.BlockSpec((1, 512), lambda i: (0, 0)),
        ],
        out_specs=pl.BlockSpec((512, 512), lambda i: (i, 0)),
        out_shape=jax.ShapeDtypeStruct((N_POINTS, 512), _F32),
    )(pos, pos1.T, _norms(pos)[:, None], _norms(pos1)[None, :], up1,
      p["seg1"][0], p["seg1"][1][None, :])

    w2p = jnp.zeros((512, 128), _F32).at[:, :13].set(p["seg2"][0])
    b2p = jnp.zeros((1, 128), _F32).at[:, :13].set(p["seg2"][1])
    logits_pad = pl.pallas_call(_head_body, out_shape=jax.ShapeDtypeStruct(
        (N_POINTS, 128), _F32))(
        h, p["bn_gamma"][None, :], p["bn_beta"][None, :], w2p, b2p)

    logits = logits_pad[:, :13]
    return jnp.transpose(logits)[None, :, :]
